# baseline (speedup 1.0000x reference)
import functools

import jax
import jax.numpy as jnp
from jax import lax
from jax.experimental import pallas as pl
from jax.experimental.pallas import tpu as pltpu
from jax.experimental.pallas import tpu_sc as plsc

TOTAL = 32768
NSEG = 16
L = 16
NC = 2
NS = 16
NW = NC * NS
TOK_W = TOTAL // NW
GCH = 128
NCH = TOK_W // GCH
NV = TOK_W // L


def _body(fidx_hbm, sidx_hbm, seg_hbm, ftab_hbm, stab_hbm, part_hbm,
          fidx_v, sidx_v, seg_v, fval_v, sval_v,
          acc_a, acc_w, pad_v, sem, *gsems):
    cid = lax.axis_index("c")
    sid = lax.axis_index("s")
    wid = cid * NS + sid
    base = pl.multiple_of(wid * TOK_W, TOK_W)

    cp1 = pltpu.async_copy(fidx_hbm.at[pl.ds(base, TOK_W)], fidx_v, sem)
    cp2 = pltpu.async_copy(sidx_hbm.at[pl.ds(base, TOK_W)], sidx_v, sem)
    cp3 = pltpu.async_copy(seg_hbm.at[pl.ds(base, TOK_W)], seg_v, sem)

    NSPL = 4
    H = TOK_W // NSPL
    parts = [pl.ds(q * H, H) for q in range(NSPL)]
    cp1.wait()
    gfs = [pltpu.async_copy(ftab_hbm.at[fidx_v.at[p]], fval_v.at[p], gsems[2 * q])
           for q, p in enumerate(parts)]
    cp2.wait()
    gss = [pltpu.async_copy(stab_hbm.at[sidx_v.at[p]], sval_v.at[p], gsems[2 * q + 1])
           for q, p in enumerate(parts)]

    acc_a[...] = jnp.zeros((L,), jnp.float32)
    acc_w[...] = jnp.zeros((L,), jnp.float32)
    cp3.wait()
    lane = lax.iota(jnp.int32, L)
    per_q = NV // NSPL
    for c in range(NV):
        if c % per_q == 0:
            q = c // per_q
            gfs[q].wait()
            gss[q].wait()
        sl = pl.ds(c * L, L)
        attn = jnp.exp(fval_v[sl])
        w = attn * sval_v[sl]
        seg = seg_v[sl]
        seg0 = seg[0]
        uniform = seg0 == seg[L - 1]

        @pl.when(uniform)
        def _():
            sa = jnp.sum(attn)
            sw = jnp.sum(w)
            hot = lane == seg0
            acc_a[...] = acc_a[...] + jnp.where(hot, sa, jnp.float32(0))
            acc_w[...] = acc_w[...] + jnp.where(hot, sw, jnp.float32(0))

        @pl.when(jnp.logical_not(uniform))
        def _():
            plsc.addupdate_scatter(acc_a, [seg], attn)
            plsc.addupdate_scatter(acc_w, [seg], w)

    pad_v[pl.ds(0, L)] = acc_a[...]
    pad_v[pl.ds(L, L)] = acc_w[...]
    pltpu.sync_copy(pad_v, part_hbm.at[wid])


@functools.lru_cache(maxsize=1)
def _make_fragnet():
    return functools.partial(
        pl.kernel,
        mesh=plsc.VectorSubcoreMesh(core_axis_name="c", subcore_axis_name="s",
                                    num_cores=NC),
        out_type=jax.ShapeDtypeStruct((NW, 128), jnp.float32),
        compiler_params=pltpu.CompilerParams(
            needs_layout_passes=False,
            skip_device_barrier=True,
            disable_bounds_checks=True,
            disable_semaphore_checks=True,
        ),
        scratch_types=[
            pltpu.VMEM((TOK_W,), jnp.int32),
            pltpu.VMEM((TOK_W,), jnp.int32),
            pltpu.VMEM((TOK_W,), jnp.int32),
            pltpu.VMEM((TOK_W,), jnp.float32),
            pltpu.VMEM((TOK_W,), jnp.float32),
            pltpu.VMEM((L,), jnp.float32),
            pltpu.VMEM((L,), jnp.float32),
            pltpu.VMEM((128,), jnp.float32),
        ] + [pltpu.SemaphoreType.DMA] * 9,
    )(_body)


def _combine_body(part_ref, bias_ref, out_ref):
    pa = jnp.sum(part_ref[:, :L], axis=0)
    pw = jnp.sum(part_ref[:, L:2 * L], axis=0)
    out_ref[...] = pw / (pa + jnp.float32(0.001)) + bias_ref[0]


def _combine(partials, bias):
    return pl.pallas_call(
        _combine_body,
        out_shape=jax.ShapeDtypeStruct((NSEG,), jnp.float32),
    )(partials, bias)


def kernel(vectors, segment_ids, frag_table, site_table, bias):
    fidx = vectors[:, 1]
    sidx = vectors[:, 0]
    ftab = jnp.reshape(frag_table, (-1,))
    stab = jnp.reshape(site_table, (-1,))
    partials = _make_fragnet()(fidx, sidx, segment_ids, ftab, stab)
    return _combine(partials, bias)
